# baseline (speedup 1.0000x reference)
import functools
import math

import jax
import jax.numpy as jnp
from jax import lax
from jax.experimental import pallas as pl
from jax.experimental.pallas import tpu as pltpu
from jax.experimental.pallas import tpu_sc as plsc

DIM = 64
LANES = 16

_NC = 2
_NS = 16
_NW = _NC * _NS

_CHUNK = 128
_BATCH = 64
_PE_W = 216


def _rsqrt_vec(x):
    i = plsc.bitcast(x, jnp.int32)
    i = jnp.full((LANES,), 0x5F3759DF, jnp.int32) - lax.shift_right_logical(i, 1)
    r = plsc.bitcast(i, jnp.float32)
    hx = 0.5 * x
    for _ in range(3):
        r = r * (1.5 - hx * r * r)
    return r


def _fused_embed_ln(table, idx, pe_t, gamma, beta, n_rows, seq_len):
    per_w = n_rows // _NW
    n_chunks = per_w // _CHUNK

    mesh = plsc.VectorSubcoreMesh(
        core_axis_name="c", subcore_axis_name="s",
        num_cores=_NC, num_subcores=_NS)

    @functools.partial(
        pl.kernel,
        mesh=mesh,
        out_type=jax.ShapeDtypeStruct((n_rows, DIM), jnp.float32),
        scratch_types=[
            pltpu.VMEM((2, _CHUNK), jnp.int32),
            pltpu.VMEM((2, _CHUNK, DIM), jnp.float32),
            pltpu.VMEM((2, _CHUNK, DIM), jnp.float32),
            pltpu.VMEM((_CHUNK,), jnp.float32),
            pltpu.VMEM((_CHUNK,), jnp.float32),
            pltpu.VMEM(pe_t.shape, jnp.float32),
            pltpu.VMEM((DIM,), jnp.float32),
            pltpu.VMEM((DIM,), jnp.float32),
            pltpu.SemaphoreType.DMA,
            pltpu.SemaphoreType.DMA,
        ],
        compiler_params=pltpu.CompilerParams(
            use_tc_tiling_on_sc=True, needs_layout_passes=False),
    )
    def k(table_hbm, idx_hbm, pe_hbm, g_hbm, b_hbm, out_hbm,
          idx_v, rows_v, stage_v, mu_v, rs_v, pe_v, g_v, b_v, sem_g, sem_o):
        wid = lax.axis_index("s") * _NC + lax.axis_index("c")
        wbase = wid * per_w

        pltpu.sync_copy(pe_hbm, pe_v)
        pltpu.sync_copy(g_hbm, g_v)
        pltpu.sync_copy(b_hbm, b_v)

        g_regs = [g_v[pl.ds(LANES * t, LANES)] for t in range(DIM // LANES)]
        b_regs = [b_v[pl.ds(LANES * t, LANES)] for t in range(DIM // LANES)]
        lane_iota = lax.iota(jnp.int32, LANES)

        def fire_gather(c):
            p = lax.rem(c, 2)
            base = wbase + c * _CHUNK
            pltpu.sync_copy(idx_hbm.at[pl.ds(base, _CHUNK)], idx_v.at[p])

            def batch(bi, _):
                def fire(j, _):
                    w = idx_v[p, pl.ds(bi * _BATCH + j * LANES, LANES)]
                    for t in range(LANES):
                        pltpu.async_copy(
                            table_hbm.at[pl.ds(w[t], 1), :],
                            rows_v.at[p, pl.ds(bi * _BATCH + j * LANES + t, 1), :],
                            sem_g)
                    return ()

                lax.fori_loop(0, _BATCH // LANES, fire, (), unroll=False)

                def drain(j, _):
                    pltpu.make_async_copy(
                        table_hbm.at[pl.ds(0, 1), :],
                        rows_v.at[p, pl.ds(j, 1), :],
                        sem_g).wait()
                    return ()

                lax.fori_loop(0, _BATCH, drain, (), unroll=False)
                return ()

            lax.fori_loop(0, _CHUNK // _BATCH, batch, (), unroll=False)

        def compute(c):
            p = lax.rem(c, 2)
            base = wbase + c * _CHUNK

            def stats_body(gi, _):
                l0 = lax.rem(base + gi * LANES, seq_len)
                rows16 = gi * LANES + lane_iota
                acc_s = jnp.zeros((LANES,), jnp.float32)
                acc_q = jnp.zeros((LANES,), jnp.float32)
                for d in range(DIM):
                    v = plsc.load_gather(
                        rows_v.at[p], [rows16, jnp.full((LANES,), d, jnp.int32)])
                    v = v + pe_v[pl.ds(d * _PE_W + l0, LANES)]
                    acc_s = acc_s + v
                    acc_q = acc_q + v * v
                mu = acc_s * (1.0 / DIM)
                var = acc_q * (1.0 / DIM) - mu * mu
                mu_v[pl.ds(gi * LANES, LANES)] = mu
                rs_v[pl.ds(gi * LANES, LANES)] = _rsqrt_vec(var + 1e-5)
                return ()

            lax.fori_loop(0, _CHUNK // LANES, stats_body, (), unroll=False)

            def norm_body(gi, _):
                muw = mu_v[pl.ds(gi * LANES, LANES)]
                rsw = rs_v[pl.ds(gi * LANES, LANES)]
                for t in range(LANES):
                    r = gi * LANES + t
                    l = lax.rem(base + r, seq_len)
                    mu = muw[t]
                    rstd = rsw[t]
                    for u in range(DIM // LANES):
                        pe16 = plsc.load_gather(
                            pe_v, [(u * LANES + lane_iota) * _PE_W
                                   + jnp.full((LANES,), l, jnp.int32)])
                        e = rows_v[p, r, pl.ds(LANES * u, LANES)] + pe16
                        stage_v[p, r, pl.ds(LANES * u, LANES)] = (
                            (e - mu) * rstd * g_regs[u] + b_regs[u])
                return ()

            lax.fori_loop(0, _CHUNK // LANES, norm_body, (), unroll=False)
            pltpu.async_copy(
                stage_v.at[p], out_hbm.at[pl.ds(base, _CHUNK)], sem_o)

        def drain_out():
            pltpu.make_async_copy(
                stage_v.at[0], out_hbm.at[pl.ds(wbase, _CHUNK)],
                sem_o).wait()

        def chunk_body(c, _):
            @pl.when(c + 1 < n_chunks)
            def _():
                fire_gather(c + 1)

            @pl.when(c >= 2)
            def _():
                drain_out()

            compute(c)
            return ()

        fire_gather(0)
        lax.fori_loop(0, n_chunks, chunk_body, (), unroll=False)
        drain_out()
        drain_out()

    return k(table, idx, pe_t, gamma, beta)


def _pe_table(length, d):
    position = jnp.arange(length, dtype=jnp.float32)[:, None]
    div_term = jnp.exp(
        jnp.arange(0, d, 2, dtype=jnp.float32) * (-math.log(10000.0) / d))
    ang = position * div_term
    return jnp.stack([jnp.sin(ang), jnp.cos(ang)], axis=-1).reshape(length, d)


def kernel(x, word_embeddings_weight, ln_gamma, ln_beta):
    b, l = x.shape
    n = b * l
    pe = _pe_table(l, DIM)
    pe_t = jnp.concatenate([pe.T, pe.T[:, :LANES]], axis=1).reshape(-1)
    out = _fused_embed_ln(word_embeddings_weight, x.reshape(n), pe_t,
                          ln_gamma, ln_beta, n, l)
    return out.reshape(b, l, DIM)
